# Initial kernel scaffold; baseline (speedup 1.0000x reference)
import functools

import jax
import jax.numpy as jnp
from jax import lax
from jax.experimental import pallas as pl
from jax.experimental.pallas import tpu as pltpu
from jax.experimental.pallas import tpu_sc as plsc

_NC = 2
_NS = 16
_NW = _NC * _NS


def _sc_gather(table, idx):
    n_idx, d = idx.shape[0], table.shape[1]
    per_w = n_idx // _NW
    n_chunks = 2
    c = per_w // n_chunks

    mesh = plsc.VectorSubcoreMesh(core_axis_name="c", subcore_axis_name="s")

    @functools.partial(
        pl.kernel,
        mesh=mesh,
        out_type=jax.ShapeDtypeStruct((n_idx, d), jnp.float32),
        scratch_types=[
            pltpu.VMEM((n_chunks, c), jnp.int32),
            pltpu.VMEM((c, d), jnp.float32),
            pltpu.SemaphoreType.DMA,
        ],
    )
    def gather_kernel(table_hbm, idx_hbm, out_hbm, idx_v, rows_v, sem):
        wid = lax.axis_index("s") * _NC + lax.axis_index("c")
        base = wid * per_w
        pltpu.sync_copy(
            idx_hbm.at[pl.ds(base, per_w)],
            idx_v.reshape(per_w),
        )
        for j in range(n_chunks):
            pltpu.async_copy(table_hbm.at[idx_v.at[j]], rows_v, sem).wait()
            pltpu.sync_copy(rows_v, out_hbm.at[pl.ds(base + j * c, c)])

    return gather_kernel(table, idx)


def _mlp_body(x_ref, w0_ref, b0_ref, w1_ref, b1_ref, o_ref):
    h = jnp.tanh(
        jnp.dot(x_ref[...], w0_ref[...], preferred_element_type=jnp.float32)
        + b0_ref[...]
    )
    logits = (
        jnp.dot(h, w1_ref[...], preferred_element_type=jnp.float32) + b1_ref[...]
    )
    shifted = logits - jnp.max(logits, axis=1, keepdims=True)
    lse = jnp.log(jnp.sum(jnp.exp(shifted), axis=1, keepdims=True))
    o_ref[...] = shifted - lse


def _mlp(x, w0, b0, w1, b1):
    b, k = x.shape
    h = w0.shape[1]
    t = w1.shape[1]
    bm = 2048
    return pl.pallas_call(
        _mlp_body,
        grid=(b // bm,),
        in_specs=[
            pl.BlockSpec((bm, k), lambda i: (i, 0)),
            pl.BlockSpec((k, h), lambda i: (0, 0)),
            pl.BlockSpec((1, h), lambda i: (0, 0)),
            pl.BlockSpec((h, t), lambda i: (0, 0)),
            pl.BlockSpec((1, t), lambda i: (0, 0)),
        ],
        out_specs=pl.BlockSpec((bm, t), lambda i: (i, 0)),
        out_shape=jax.ShapeDtypeStruct((b, t), jnp.float32),
    )(x, w0, b0.reshape(1, h), w1, b1.reshape(1, t))


def kernel(v, E, W0, b0, W1, b1):
    batch, win = v.shape
    emb = E.shape[1]
    idx = v.reshape(batch * win)
    rows = _sc_gather(E, idx)
    x = rows.reshape(batch, win * emb)
    return _mlp(x, W0, b0, W1, b1)


# speedup vs baseline: 3.2328x; 3.2328x over previous
import functools

import jax
import jax.numpy as jnp
from jax import lax
from jax.experimental import pallas as pl
from jax.experimental.pallas import tpu as pltpu
from jax.experimental.pallas import tpu_sc as plsc

_NC = 2
_NS = 16
_NW = _NC * _NS


def _sc_gather(table, idx):
    nw, n_chunks, lane = idx.shape
    n_idx = nw * n_chunks * lane
    d = table.shape[1]

    mesh = plsc.VectorSubcoreMesh(core_axis_name="c", subcore_axis_name="s")

    @functools.partial(
        pl.kernel,
        mesh=mesh,
        out_type=jax.ShapeDtypeStruct((n_idx, d), jnp.float32),
        scratch_types=[
            pltpu.VMEM((n_chunks, lane), jnp.int32),
            pltpu.VMEM((2, lane, d), jnp.float32),
            pltpu.SemaphoreType.DMA,
            pltpu.SemaphoreType.DMA,
            pltpu.SemaphoreType.DMA,
            pltpu.SemaphoreType.DMA,
        ],
    )
    def gather_kernel(table_hbm, idx_hbm, out_hbm, idx_v, rows_v, gs0, gs1, os0, os1):
        wid = lax.axis_index("s") * _NC + lax.axis_index("c")
        per_w = n_chunks * lane
        base = wid * per_w
        pltpu.sync_copy(idx_hbm.at[wid], idx_v)
        gsems = (gs0, gs1)
        osems = (os0, os1)
        gathers = [None] * n_chunks
        outs = [None] * n_chunks
        gathers[0] = pltpu.async_copy(
            table_hbm.at[idx_v.at[0]], rows_v.at[0], gsems[0]
        )
        for j in range(n_chunks):
            b = j % 2
            nb = (j + 1) % 2
            if j + 1 < n_chunks:
                if outs[j - 1] is not None:
                    outs[j - 1].wait()
                gathers[j + 1] = pltpu.async_copy(
                    table_hbm.at[idx_v.at[j + 1]], rows_v.at[nb], gsems[nb]
                )
            gathers[j].wait()
            outs[j] = pltpu.async_copy(
                rows_v.at[b], out_hbm.at[pl.ds(base + j * lane, lane)], osems[b]
            )
        outs[n_chunks - 2].wait()
        outs[n_chunks - 1].wait()

    return gather_kernel(table, idx)


def _make_mlp(win, emb, bm):
    def mlp_body(*refs):
        x_refs = refs[:win]
        w0_ref, b0_ref, w1_ref, b1_ref, o_ref = refs[win:]
        acc = jnp.broadcast_to(b0_ref[...], (bm, b0_ref.shape[1])).astype(jnp.float32)
        for w in range(win):
            acc = acc + jnp.dot(
                x_refs[w][:, :emb],
                w0_ref[w * emb:(w + 1) * emb, :],
                preferred_element_type=jnp.float32,
            )
        h = jnp.tanh(acc)
        logits = (
            jnp.dot(h, w1_ref[...], preferred_element_type=jnp.float32)
            + b1_ref[...]
        )
        shifted = logits - jnp.max(logits, axis=1, keepdims=True)
        lse = jnp.log(jnp.sum(jnp.exp(shifted), axis=1, keepdims=True))
        o_ref[...] = shifted - lse

    return mlp_body


def _mlp(rows, batch, win, w0, b0, w1, b1):
    emb = w0.shape[0] // win
    h1 = w0.shape[1]
    t = w1.shape[1]
    d = rows.shape[1]
    bm = 2048
    nb = batch // bm
    x_specs = [
        pl.BlockSpec((bm, d), functools.partial(lambda w, i: (w * nb + i, 0), w))
        for w in range(win)
    ]
    return pl.pallas_call(
        _make_mlp(win, emb, bm),
        grid=(nb,),
        in_specs=x_specs + [
            pl.BlockSpec((win * emb, h1), lambda i: (0, 0)),
            pl.BlockSpec((1, h1), lambda i: (0, 0)),
            pl.BlockSpec((h1, t), lambda i: (0, 0)),
            pl.BlockSpec((1, t), lambda i: (0, 0)),
        ],
        out_specs=pl.BlockSpec((bm, t), lambda i: (i, 0)),
        out_shape=jax.ShapeDtypeStruct((batch, t), jnp.float32),
    )(*([rows] * win), w0, b0.reshape(1, h1), w1, b1.reshape(1, t))


def kernel(v, E, W0, b0, W1, b1):
    batch, win = v.shape
    vocab, emb = E.shape
    d = 128
    e_pad = jnp.pad(E, ((0, 0), (0, d - emb)))
    idx = v.T.reshape(_NW, batch * win // (_NW * 128), 128)
    rows = _sc_gather(e_pad, idx)
    return _mlp(rows, batch, win, W0, b0, W1, b1)
